# Initial kernel scaffold; baseline (speedup 1.0000x reference)
import jax
import jax.numpy as jnp
from jax.experimental import pallas as pl


def kernel(feat_l, feat_r, edge_lr, edge_rl, edge_ll, edge_rr, W_lr_0, W_ll_0, self_W_0, self_b_0, W_lr_1, W_ll_1, self_W_1, self_b_1):
    raise NotImplementedError("write your pallas kernel here")


# speedup vs baseline: 3.4069x; 3.4069x over previous
import functools

import jax
import jax.numpy as jnp
from jax import lax
from jax.experimental import pallas as pl
from jax.experimental.pallas import tpu as pltpu
from jax.experimental.pallas import tpu_sc as plsc

N = 25000
D = 128
DH = 64
E = 150000
B = 128
EP = E + 16
NB = EP // B
NTILES = 16
N_PAD = 25088
RPT = N_PAD // NTILES
ZCH = 196
TRASH = 25024
BT = 1000


def _agg_body(fl0, fl1, fr0, fr1,
              src_lr, dst_lr, src_rl, dst_rl,
              src_ll, dst_ll, src_rr, dst_rr, zeros_h,
              s_lr0, s_lr1, s_rl0, s_rl1, s_ll0, s_ll1, s_rr0, s_rr1,
              spm, zbuf, src_v, dst_v, rows_v, sem):
    c = lax.axis_index("c")
    s = lax.axis_index("s")
    pltpu.sync_copy(zeros_h, zbuf)

    def one_etype(feat, src_h, dst_h, out):
        for j in range(RPT // ZCH):
            pltpu.sync_copy(zbuf, spm.at[pl.ds(s * RPT + j * ZCH, ZCH)])
        plsc.subcore_barrier()
        nb = (NB - s + NTILES - 1) // NTILES

        def body(k, carry):
            off = pl.multiple_of((s + NTILES * k) * B, B)
            pltpu.sync_copy(src_h.at[pl.ds(off, B)], src_v)
            pltpu.sync_copy(dst_h.at[pl.ds(off, B)], dst_v)
            pltpu.async_copy(feat.at[src_v], rows_v, sem).wait()
            pltpu.sync_copy(rows_v, spm.at[dst_v], add=True)
            return carry

        lax.fori_loop(0, nb, body, 0)
        plsc.subcore_barrier()
        row0 = pl.multiple_of(s * RPT, RPT)
        pltpu.sync_copy(spm.at[pl.ds(row0, RPT)], out.at[pl.ds(row0, RPT)])
        plsc.subcore_barrier()

    def half(flh, frh, olr, orl, oll, orr):
        one_etype(flh, src_lr, dst_lr, olr)
        one_etype(frh, src_rl, dst_rl, orl)
        one_etype(flh, src_ll, dst_ll, oll)
        one_etype(frh, src_rr, dst_rr, orr)

    @pl.when(c == 0)
    def _():
        half(fl0, fr0, s_lr0, s_rl0, s_ll0, s_rr0)

    @pl.when(c == 1)
    def _():
        half(fl1, fr1, s_lr1, s_rl1, s_ll1, s_rr1)


def _run_agg(fl0, fl1, fr0, fr1, edges, zeros_h):
    mesh = plsc.VectorSubcoreMesh(core_axis_name="c", subcore_axis_name="s")
    out_type = [jax.ShapeDtypeStruct((N_PAD, DH), jnp.float32)] * 8
    scratch = [
        pltpu.VMEM_SHARED((N_PAD, DH), jnp.float32),
        pltpu.VMEM((ZCH, DH), jnp.float32),
        pltpu.VMEM((B,), jnp.int32),
        pltpu.VMEM((B,), jnp.int32),
        pltpu.VMEM((B, DH), jnp.float32),
        pltpu.SemaphoreType.DMA,
    ]
    fn = pl.kernel(_agg_body, out_type=out_type, mesh=mesh,
                   scratch_types=scratch,
                   compiler_params=pltpu.CompilerParams(
                       use_tc_tiling_on_sc=False))
    return fn(fl0, fl1, fr0, fr1, *edges, zeros_h)


def _counts_body(dst_lr, dst_rl, dst_ll, dst_rr, ones_h, zeros_h,
                 c_lr, c_rl, c_ll, c_rr,
                 spm, ones_v, zbuf, dst_v):
    c = lax.axis_index("c")
    s = lax.axis_index("s")
    pltpu.sync_copy(ones_h, ones_v)
    pltpu.sync_copy(zeros_h, zbuf)

    def one(dst_h, out):
        row0 = pl.multiple_of(s * RPT, RPT)
        pltpu.sync_copy(zbuf, spm.at[pl.ds(row0, RPT)])
        plsc.subcore_barrier()
        nb = (NB - s + NTILES - 1) // NTILES

        def body(k, carry):
            off = pl.multiple_of((s + NTILES * k) * B, B)
            pltpu.sync_copy(dst_h.at[pl.ds(off, B)], dst_v)
            pltpu.sync_copy(ones_v, spm.at[dst_v], add=True)
            return carry

        lax.fori_loop(0, nb, body, 0)
        plsc.subcore_barrier()
        pltpu.sync_copy(spm.at[pl.ds(row0, RPT)], out.at[pl.ds(row0, RPT)])
        plsc.subcore_barrier()

    @pl.when(c == 0)
    def _():
        one(dst_lr, c_lr)
        one(dst_rl, c_rl)

    @pl.when(c == 1)
    def _():
        one(dst_ll, c_ll)
        one(dst_rr, c_rr)


def _run_counts(dst_lr, dst_rl, dst_ll, dst_rr):
    mesh = plsc.VectorSubcoreMesh(core_axis_name="c", subcore_axis_name="s")
    out_type = [jax.ShapeDtypeStruct((N_PAD, 16), jnp.float32)] * 4
    scratch = [
        pltpu.VMEM_SHARED((N_PAD, 16), jnp.float32),
        pltpu.VMEM((B, 16), jnp.float32),
        pltpu.VMEM((RPT, 16), jnp.float32),
        pltpu.VMEM((B,), jnp.int32),
    ]
    ones_h = jnp.ones((B, 16), jnp.float32)
    zeros_h = jnp.zeros((RPT, 16), jnp.float32)
    fn = pl.kernel(_counts_body, out_type=out_type, mesh=mesh,
                   scratch_types=scratch,
                   compiler_params=pltpu.CompilerParams(
                       use_tc_tiling_on_sc=False))
    return fn(dst_lr, dst_rl, dst_ll, dst_rr, ones_h, zeros_h)


def _dense_block(split, sA0, sA1, cA, sB0, sB1, cB, f0, f1,
                 wA, wB, wS, bias, *outs):
    sA = jnp.concatenate([sA0[...], sA1[...]], axis=1)
    sB = jnp.concatenate([sB0[...], sB1[...]], axis=1)
    invA = 1.0 / jnp.maximum(cA[...][:, 0:1], 1.0)
    invB = 1.0 / jnp.maximum(cB[...][:, 0:1], 1.0)
    inter = jnp.dot(sA * invA, wA[...], preferred_element_type=jnp.float32)
    intra = jnp.dot(sB * invB, wB[...], preferred_element_type=jnp.float32)
    f = jnp.concatenate([f0[...], f1[...]], axis=1)
    self_t = jnp.dot(f, wS[...], preferred_element_type=jnp.float32)
    y = jnp.maximum((inter + intra) * 0.5, 0.0) + self_t + bias[...]
    if split:
        outs[0][...] = y[:, :DH]
        outs[1][...] = y[:, DH:]
    else:
        outs[0][...] = y


def _run_dense(sA0, sA1, cA, sB0, sB1, cB, f0, f1, wA, wB, wS, b, split):
    half_spec = pl.BlockSpec((BT, DH), lambda i: (i, 0))
    cnt_spec = pl.BlockSpec((BT, 16), lambda i: (i, 0))
    w_spec = pl.BlockSpec((D, D), lambda i: (0, 0))
    b_spec = pl.BlockSpec((1, D), lambda i: (0, 0))
    if split:
        out_shape = [jax.ShapeDtypeStruct((N, DH), jnp.float32)] * 2
        out_specs = [half_spec, half_spec]
    else:
        out_shape = [jax.ShapeDtypeStruct((N, D), jnp.float32)]
        out_specs = [pl.BlockSpec((BT, D), lambda i: (i, 0))]
    res = pl.pallas_call(
        functools.partial(_dense_block, split),
        grid=(N // BT,),
        in_specs=[half_spec, half_spec, cnt_spec, half_spec, half_spec,
                  cnt_spec, half_spec, half_spec, w_spec, w_spec, w_spec,
                  b_spec],
        out_specs=out_specs,
        out_shape=out_shape,
    )(sA0, sA1, cA, sB0, sB1, cB, f0, f1, wA, wB, wS, b.reshape(1, D))
    return res if split else res[0]


def kernel(feat_l, feat_r, edge_lr, edge_rl, edge_ll, edge_rr,
           W_lr_0, W_ll_0, self_W_0, self_b_0,
           W_lr_1, W_ll_1, self_W_1, self_b_1):
    fl0, fl1 = feat_l[:, :DH], feat_l[:, DH:]
    fr0, fr1 = feat_r[:, :DH], feat_r[:, DH:]

    pad_src = jnp.zeros((EP - E,), jnp.int32)
    pad_dst = jnp.full((EP - E,), TRASH, jnp.int32)

    def pad(e):
        return (jnp.concatenate([e[0], pad_src]),
                jnp.concatenate([e[1], pad_dst]))

    src_lr, dst_lr = pad(edge_lr)
    src_rl, dst_rl = pad(edge_rl)
    src_ll, dst_ll = pad(edge_ll)
    src_rr, dst_rr = pad(edge_rr)
    edges = (src_lr, dst_lr, src_rl, dst_rl, src_ll, dst_ll, src_rr, dst_rr)

    c_lr, c_rl, c_ll, c_rr = _run_counts(dst_lr, dst_rl, dst_ll, dst_rr)

    zeros_h = jnp.zeros((ZCH, DH), jnp.float32)

    (s_lr0, s_lr1, s_rl0, s_rl1,
     s_ll0, s_ll1, s_rr0, s_rr1) = _run_agg(fl0, fl1, fr0, fr1, edges,
                                            zeros_h)

    nl0, nl1 = _run_dense(s_rl0, s_rl1, c_rl, s_ll0, s_ll1, c_ll,
                          fl0, fl1, W_lr_0, W_ll_0, self_W_0, self_b_0,
                          split=True)
    nr0, nr1 = _run_dense(s_lr0, s_lr1, c_lr, s_rr0, s_rr1, c_rr,
                          fr0, fr1, W_lr_0, W_ll_0, self_W_0, self_b_0,
                          split=True)

    (t_lr0, t_lr1, t_rl0, t_rl1,
     t_ll0, t_ll1, t_rr0, t_rr1) = _run_agg(nl0, nl1, nr0, nr1, edges,
                                            zeros_h)

    nl = _run_dense(t_rl0, t_rl1, c_rl, t_ll0, t_ll1, c_ll,
                    nl0, nl1, W_lr_1, W_ll_1, self_W_1, self_b_1,
                    split=False)
    nr = _run_dense(t_lr0, t_lr1, c_lr, t_rr0, t_rr1, c_rr,
                    nr0, nr1, W_lr_1, W_ll_1, self_W_1, self_b_1,
                    split=False)

    return jnp.concatenate([nl, nr], axis=0)


# speedup vs baseline: 4.1518x; 1.2186x over previous
import functools

import jax
import jax.numpy as jnp
from jax import lax
from jax.experimental import pallas as pl
from jax.experimental.pallas import tpu as pltpu
from jax.experimental.pallas import tpu_sc as plsc

N = 25000
D = 128
DH = 64
E = 150000
B = 128
NTILES = 16
NBT = 74
NB = NBT * NTILES
EP = NB * B
N_PAD = 25088
RPT = N_PAD // NTILES
TRASH = 25024
BT = 1000


def _agg_body(fl0, fl1, fr0, fr1,
              src_lr, dst_lr, src_rl, dst_rl,
              src_ll, dst_ll, src_rr, dst_rr, zeros_h,
              s_lr0, s_lr1, s_rl0, s_rl1, s_ll0, s_ll1, s_rr0, s_rr1,
              spm, dst_all, sv0, sv1, r0, r1,
              gs0, gs1, ss0, ss1, is0, is1):
    c = lax.axis_index("c")
    s = lax.axis_index("s")
    sv = (sv0, sv1)
    rr = (r0, r1)
    gs = (gs0, gs1)
    ss = (ss0, ss1)
    isem = (is0, is1)

    def one_etype(feat, src_h, dst_h, out):
        row0 = pl.multiple_of(s * RPT, RPT)
        pltpu.sync_copy(zeros_h, spm.at[pl.ds(row0, RPT)])
        plsc.subcore_barrier()
        base = pl.multiple_of(s * NBT, 2)
        pltpu.sync_copy(dst_h.at[pl.ds(base, NBT)], dst_all)
        pltpu.sync_copy(src_h.at[base], sv0)
        pltpu.async_copy(src_h.at[base + 1], sv1, is1)
        pltpu.async_copy(feat.at[sv0], r0, gs0)

        def step(k, k2, p):
            q = 1 - p
            pltpu.make_async_copy(feat.at[sv[p]], rr[p], gs[p]).wait()
            pltpu.async_copy(rr[p], spm.at[dst_all.at[k]], ss[p], add=True)

            @pl.when(k + 2 < NBT)
            def _():
                pltpu.async_copy(src_h.at[base + k + 2], sv[p], isem[p])

            @pl.when(k > 0)
            def _():
                pltpu.make_async_copy(rr[q], spm.at[dst_all.at[0]],
                                      ss[q]).wait()

            @pl.when(k + 1 < NBT)
            def _():
                pltpu.make_async_copy(src_h.at[base], sv[q], isem[q]).wait()
                pltpu.async_copy(feat.at[sv[q]], rr[q], gs[q])

        def body(k2, carry):
            step(2 * k2, k2, 0)
            step(2 * k2 + 1, k2, 1)
            return carry

        lax.fori_loop(0, NBT // 2, body, 0)
        pltpu.make_async_copy(rr[1], spm.at[dst_all.at[0]], ss[1]).wait()
        plsc.subcore_barrier()
        pltpu.sync_copy(spm.at[pl.ds(row0, RPT)], out.at[pl.ds(row0, RPT)])
        plsc.subcore_barrier()

    def half(flh, frh, olr, orl, oll, orr):
        one_etype(flh, src_lr, dst_lr, olr)
        one_etype(frh, src_rl, dst_rl, orl)
        one_etype(flh, src_ll, dst_ll, oll)
        one_etype(frh, src_rr, dst_rr, orr)

    @pl.when(c == 0)
    def _():
        half(fl0, fr0, s_lr0, s_rl0, s_ll0, s_rr0)

    @pl.when(c == 1)
    def _():
        half(fl1, fr1, s_lr1, s_rl1, s_ll1, s_rr1)


def _run_agg(fl0, fl1, fr0, fr1, edges, zeros_h):
    mesh = plsc.VectorSubcoreMesh(core_axis_name="c", subcore_axis_name="s")
    out_type = [jax.ShapeDtypeStruct((N_PAD, DH), jnp.float32)] * 8
    scratch = [
        pltpu.VMEM_SHARED((N_PAD, DH), jnp.float32),
        pltpu.VMEM((NBT, B), jnp.int32),
        pltpu.VMEM((B,), jnp.int32),
        pltpu.VMEM((B,), jnp.int32),
        pltpu.VMEM((B, DH), jnp.float32),
        pltpu.VMEM((B, DH), jnp.float32),
        pltpu.SemaphoreType.DMA,
        pltpu.SemaphoreType.DMA,
        pltpu.SemaphoreType.DMA,
        pltpu.SemaphoreType.DMA,
        pltpu.SemaphoreType.DMA,
        pltpu.SemaphoreType.DMA,
    ]
    fn = pl.kernel(_agg_body, out_type=out_type, mesh=mesh,
                   scratch_types=scratch,
                   compiler_params=pltpu.CompilerParams(
                       use_tc_tiling_on_sc=False))
    return fn(fl0, fl1, fr0, fr1, *edges, zeros_h)


def _counts_body(dst_lr, dst_rl, dst_ll, dst_rr, ones_h, zeros_h,
                 c_lr, c_rl, c_ll, c_rr,
                 spm, ones_v, zbuf, dst_v):
    c = lax.axis_index("c")
    s = lax.axis_index("s")
    pltpu.sync_copy(ones_h, ones_v)
    pltpu.sync_copy(zeros_h, zbuf)

    def one(dst_h, out):
        row0 = pl.multiple_of(s * RPT, RPT)
        pltpu.sync_copy(zbuf, spm.at[pl.ds(row0, RPT)])
        plsc.subcore_barrier()
        base = s * NBT

        def body(k, carry):
            pltpu.sync_copy(dst_h.at[base + k], dst_v)
            pltpu.sync_copy(ones_v, spm.at[dst_v], add=True)
            return carry

        lax.fori_loop(0, NBT, body, 0)
        plsc.subcore_barrier()
        pltpu.sync_copy(spm.at[pl.ds(row0, RPT)], out.at[pl.ds(row0, RPT)])
        plsc.subcore_barrier()

    @pl.when(c == 0)
    def _():
        one(dst_lr, c_lr)
        one(dst_rl, c_rl)

    @pl.when(c == 1)
    def _():
        one(dst_ll, c_ll)
        one(dst_rr, c_rr)


def _run_counts(dst_lr, dst_rl, dst_ll, dst_rr):
    mesh = plsc.VectorSubcoreMesh(core_axis_name="c", subcore_axis_name="s")
    out_type = [jax.ShapeDtypeStruct((N_PAD, 16), jnp.float32)] * 4
    scratch = [
        pltpu.VMEM_SHARED((N_PAD, 16), jnp.float32),
        pltpu.VMEM((B, 16), jnp.float32),
        pltpu.VMEM((RPT, 16), jnp.float32),
        pltpu.VMEM((B,), jnp.int32),
    ]
    ones_h = jnp.ones((B, 16), jnp.float32)
    zeros_h = jnp.zeros((RPT, 16), jnp.float32)
    fn = pl.kernel(_counts_body, out_type=out_type, mesh=mesh,
                   scratch_types=scratch,
                   compiler_params=pltpu.CompilerParams(
                       use_tc_tiling_on_sc=False))
    return fn(dst_lr, dst_rl, dst_ll, dst_rr, ones_h, zeros_h)


def _dense_block(split, sA0, sA1, cA, sB0, sB1, cB, f0, f1,
                 wA, wB, wS, bias, *outs):
    sA = jnp.concatenate([sA0[...], sA1[...]], axis=1)
    sB = jnp.concatenate([sB0[...], sB1[...]], axis=1)
    invA = 1.0 / jnp.maximum(cA[...][:, 0:1], 1.0)
    invB = 1.0 / jnp.maximum(cB[...][:, 0:1], 1.0)
    inter = jnp.dot(sA * invA, wA[...], preferred_element_type=jnp.float32)
    intra = jnp.dot(sB * invB, wB[...], preferred_element_type=jnp.float32)
    f = jnp.concatenate([f0[...], f1[...]], axis=1)
    self_t = jnp.dot(f, wS[...], preferred_element_type=jnp.float32)
    y = jnp.maximum((inter + intra) * 0.5, 0.0) + self_t + bias[...]
    if split:
        outs[0][...] = y[:, :DH]
        outs[1][...] = y[:, DH:]
    else:
        outs[0][...] = y


def _run_dense(sA0, sA1, cA, sB0, sB1, cB, f0, f1, wA, wB, wS, b, split):
    half_spec = pl.BlockSpec((BT, DH), lambda i: (i, 0))
    cnt_spec = pl.BlockSpec((BT, 16), lambda i: (i, 0))
    w_spec = pl.BlockSpec((D, D), lambda i: (0, 0))
    b_spec = pl.BlockSpec((1, D), lambda i: (0, 0))
    if split:
        out_shape = [jax.ShapeDtypeStruct((N, DH), jnp.float32)] * 2
        out_specs = [half_spec, half_spec]
    else:
        out_shape = [jax.ShapeDtypeStruct((N, D), jnp.float32)]
        out_specs = [pl.BlockSpec((BT, D), lambda i: (i, 0))]
    res = pl.pallas_call(
        functools.partial(_dense_block, split),
        grid=(N // BT,),
        in_specs=[half_spec, half_spec, cnt_spec, half_spec, half_spec,
                  cnt_spec, half_spec, half_spec, w_spec, w_spec, w_spec,
                  b_spec],
        out_specs=out_specs,
        out_shape=out_shape,
    )(sA0, sA1, cA, sB0, sB1, cB, f0, f1, wA, wB, wS, b.reshape(1, D))
    return res if split else res[0]


def kernel(feat_l, feat_r, edge_lr, edge_rl, edge_ll, edge_rr,
           W_lr_0, W_ll_0, self_W_0, self_b_0,
           W_lr_1, W_ll_1, self_W_1, self_b_1):
    fl0, fl1 = feat_l[:, :DH], feat_l[:, DH:]
    fr0, fr1 = feat_r[:, :DH], feat_r[:, DH:]

    pad_src = jnp.zeros((EP - E,), jnp.int32)
    pad_dst = jnp.full((EP - E,), TRASH, jnp.int32)

    def pad(e):
        return (jnp.concatenate([e[0], pad_src]).reshape(NB, B),
                jnp.concatenate([e[1], pad_dst]).reshape(NB, B))

    src_lr, dst_lr = pad(edge_lr)
    src_rl, dst_rl = pad(edge_rl)
    src_ll, dst_ll = pad(edge_ll)
    src_rr, dst_rr = pad(edge_rr)
    edges = (src_lr, dst_lr, src_rl, dst_rl, src_ll, dst_ll, src_rr, dst_rr)

    c_lr, c_rl, c_ll, c_rr = _run_counts(dst_lr, dst_rl, dst_ll, dst_rr)

    zeros_h = jnp.zeros((RPT, DH), jnp.float32)

    (s_lr0, s_lr1, s_rl0, s_rl1,
     s_ll0, s_ll1, s_rr0, s_rr1) = _run_agg(fl0, fl1, fr0, fr1, edges,
                                            zeros_h)

    nl0, nl1 = _run_dense(s_rl0, s_rl1, c_rl, s_ll0, s_ll1, c_ll,
                          fl0, fl1, W_lr_0, W_ll_0, self_W_0, self_b_0,
                          split=True)
    nr0, nr1 = _run_dense(s_lr0, s_lr1, c_lr, s_rr0, s_rr1, c_rr,
                          fr0, fr1, W_lr_0, W_ll_0, self_W_0, self_b_0,
                          split=True)

    (t_lr0, t_lr1, t_rl0, t_rl1,
     t_ll0, t_ll1, t_rr0, t_rr1) = _run_agg(nl0, nl1, nr0, nr1, edges,
                                            zeros_h)

    nl = _run_dense(t_rl0, t_rl1, c_rl, t_ll0, t_ll1, c_ll,
                    nl0, nl1, W_lr_1, W_ll_1, self_W_1, self_b_1,
                    split=False)
    nr = _run_dense(t_lr0, t_lr1, c_lr, t_rr0, t_rr1, c_rr,
                    nr0, nr1, W_lr_1, W_ll_1, self_W_1, self_b_1,
                    split=False)

    return jnp.concatenate([nl, nr], axis=0)


# speedup vs baseline: 6.0431x; 1.4555x over previous
import functools

import jax
import jax.numpy as jnp
from jax import lax
from jax.experimental import pallas as pl
from jax.experimental.pallas import tpu as pltpu
from jax.experimental.pallas import tpu_sc as plsc

N = 25000
D = 128
DH = 64
E = 150000
B = 112
NTILES = 16
NBT = 84
NB = NBT * NTILES
EP = NB * B
N_PAD = 25088
RPT = N_PAD // NTILES
TRASH = 25024
BT = 1000


def _agg_body(fl0, fl1, fr0, fr1,
              src_lr, dst_lr, src_rl, dst_rl,
              src_ll, dst_ll, src_rr, dst_rr, zeros_h,
              s_lr0, s_lr1, s_rl0, s_rl1, s_ll0, s_ll1, s_rr0, s_rr1,
              spm, sv0, sv1, sv2, sv3, dv0, dv1, dv2, dv3,
              r0, r1, r2, r3,
              gs0, gs1, gs2, gs3, ss0, ss1, ss2, ss3,
              is0, is1, is2, is3, js0, js1, js2, js3):
    c = lax.axis_index("c")
    s = lax.axis_index("s")
    sv = (sv0, sv1, sv2, sv3)
    dv = (dv0, dv1, dv2, dv3)
    rv = (r0, r1, r2, r3)
    gs = (gs0, gs1, gs2, gs3)
    ss = (ss0, ss1, ss2, ss3)
    isem = (is0, is1, is2, is3)
    jsem = (js0, js1, js2, js3)

    def one_etype(feat, src_h, dst_h, out):
        row0 = pl.multiple_of(s * RPT, RPT)
        pltpu.sync_copy(zeros_h, spm.at[pl.ds(row0, RPT)])
        plsc.subcore_barrier()
        base = s * NBT
        for j in range(4):
            pltpu.async_copy(src_h.at[base + j], sv[j], isem[j])
        for j in range(2):
            pltpu.async_copy(dst_h.at[base + j], dv[j], jsem[j])
        for j in range(2):
            pltpu.make_async_copy(src_h.at[base], sv[j], isem[j]).wait()
            pltpu.async_copy(feat.at[sv[j]], rv[j], gs[j])

        def step(m, j):
            k = 4 * m + j
            p = j
            n = (j + 2) % 4

            def wait_scatter_km2():
                pltpu.make_async_copy(rv[n], spm.at[dv[n]], ss[n]).wait()

            def fill_n():
                pltpu.async_copy(dst_h.at[base + k + 2], dv[n], jsem[n])
                pltpu.make_async_copy(src_h.at[base], sv[n], isem[n]).wait()
                pltpu.async_copy(feat.at[sv[n]], rv[n], gs[n])

            if j < 2:
                m_gt0 = m > 0
                pl.when(m_gt0)(wait_scatter_km2)
                fill_n()
            else:
                wait_scatter_km2()
                pl.when(4 * m + j + 2 < NBT)(fill_n)
            pltpu.make_async_copy(feat.at[sv[p]], rv[p], gs[p]).wait()
            pltpu.make_async_copy(dst_h.at[base], dv[p], jsem[p]).wait()
            pltpu.async_copy(rv[p], spm.at[dv[p]], ss[p], add=True)

            @pl.when(4 * m + j + 4 < NBT)
            def _():
                pltpu.async_copy(src_h.at[base + k + 4], sv[p], isem[p])

        def body(m, carry):
            for j in range(4):
                step(m, j)
            return carry

        lax.fori_loop(0, NBT // 4, body, 0)
        for j in (2, 3):
            pltpu.make_async_copy(rv[j], spm.at[dv[j]], ss[j]).wait()
        plsc.subcore_barrier()
        pltpu.sync_copy(spm.at[pl.ds(row0, RPT)], out.at[pl.ds(row0, RPT)])
        plsc.subcore_barrier()

    def half(flh, frh, olr, orl, oll, orr):
        one_etype(flh, src_lr, dst_lr, olr)
        one_etype(frh, src_rl, dst_rl, orl)
        one_etype(flh, src_ll, dst_ll, oll)
        one_etype(frh, src_rr, dst_rr, orr)

    @pl.when(c == 0)
    def _():
        half(fl0, fr0, s_lr0, s_rl0, s_ll0, s_rr0)

    @pl.when(c == 1)
    def _():
        half(fl1, fr1, s_lr1, s_rl1, s_ll1, s_rr1)


def _run_agg(fl0, fl1, fr0, fr1, edges, zeros_h):
    mesh = plsc.VectorSubcoreMesh(core_axis_name="c", subcore_axis_name="s")
    out_type = [jax.ShapeDtypeStruct((N_PAD, DH), jnp.float32)] * 8
    scratch = (
        [pltpu.VMEM_SHARED((N_PAD, DH), jnp.float32)]
        + [pltpu.VMEM((B,), jnp.int32) for _ in range(8)]
        + [pltpu.VMEM((B, DH), jnp.float32) for _ in range(4)]
        + [pltpu.SemaphoreType.DMA for _ in range(16)]
    )
    fn = pl.kernel(_agg_body, out_type=out_type, mesh=mesh,
                   scratch_types=scratch,
                   compiler_params=pltpu.CompilerParams(
                       use_tc_tiling_on_sc=False))
    return fn(fl0, fl1, fr0, fr1, *edges, zeros_h)


def _counts_body(dst_lr, dst_rl, dst_ll, dst_rr, ones_h, zeros_h,
                 c_lr, c_rl, c_ll, c_rr,
                 spm, ones_v, zbuf, dv0, dv1, dv2, dv3,
                 ss0, ss1, ss2, ss3, js0, js1, js2, js3):
    c = lax.axis_index("c")
    s = lax.axis_index("s")
    dv = (dv0, dv1, dv2, dv3)
    ss = (ss0, ss1, ss2, ss3)
    jsem = (js0, js1, js2, js3)
    pltpu.sync_copy(ones_h, ones_v)
    pltpu.sync_copy(zeros_h, zbuf)

    def one(dst_h, out):
        row0 = pl.multiple_of(s * RPT, RPT)
        pltpu.sync_copy(zbuf, spm.at[pl.ds(row0, RPT)])
        plsc.subcore_barrier()
        base = s * NBT
        for j in range(2):
            pltpu.async_copy(dst_h.at[base + j], dv[j], jsem[j])

        def step(m, j):
            k = 4 * m + j
            p = j
            n = (j + 2) % 4

            def refill():
                pltpu.make_async_copy(ones_v, spm.at[dv[n]], ss[n]).wait()
                pltpu.async_copy(dst_h.at[base + k + 2], dv[n], jsem[n])

            if j < 2:
                pl.when(m > 0)(lambda: pltpu.make_async_copy(
                    ones_v, spm.at[dv[n]], ss[n]).wait())
                pltpu.async_copy(dst_h.at[base + k + 2], dv[n], jsem[n])
            else:
                pl.when(4 * m + j + 2 < NBT)(refill)
                pl.when(4 * m + j + 2 >= NBT)(lambda: pltpu.make_async_copy(
                    ones_v, spm.at[dv[n]], ss[n]).wait())
            pltpu.make_async_copy(dst_h.at[base], dv[p], jsem[p]).wait()
            pltpu.async_copy(ones_v, spm.at[dv[p]], ss[p], add=True)

        def body(m, carry):
            for j in range(4):
                step(m, j)
            return carry

        lax.fori_loop(0, NBT // 4, body, 0)
        for j in (2, 3):
            pltpu.make_async_copy(ones_v, spm.at[dv[j]], ss[j]).wait()
        plsc.subcore_barrier()
        pltpu.sync_copy(spm.at[pl.ds(row0, RPT)], out.at[pl.ds(row0, RPT)])
        plsc.subcore_barrier()

    @pl.when(c == 0)
    def _():
        one(dst_lr, c_lr)
        one(dst_rl, c_rl)

    @pl.when(c == 1)
    def _():
        one(dst_ll, c_ll)
        one(dst_rr, c_rr)


def _run_counts(dst_lr, dst_rl, dst_ll, dst_rr):
    mesh = plsc.VectorSubcoreMesh(core_axis_name="c", subcore_axis_name="s")
    out_type = [jax.ShapeDtypeStruct((N_PAD, 16), jnp.float32)] * 4
    scratch = (
        [pltpu.VMEM_SHARED((N_PAD, 16), jnp.float32),
         pltpu.VMEM((B, 16), jnp.float32),
         pltpu.VMEM((RPT, 16), jnp.float32)]
        + [pltpu.VMEM((B,), jnp.int32) for _ in range(4)]
        + [pltpu.SemaphoreType.DMA for _ in range(8)]
    )
    ones_h = jnp.ones((B, 16), jnp.float32)
    zeros_h = jnp.zeros((RPT, 16), jnp.float32)
    fn = pl.kernel(_counts_body, out_type=out_type, mesh=mesh,
                   scratch_types=scratch,
                   compiler_params=pltpu.CompilerParams(
                       use_tc_tiling_on_sc=False))
    return fn(dst_lr, dst_rl, dst_ll, dst_rr, ones_h, zeros_h)


def _dense_block(split, sA0, sA1, cA, sB0, sB1, cB, f0, f1,
                 wA, wB, wS, bias, *outs):
    sA = jnp.concatenate([sA0[...], sA1[...]], axis=1)
    sB = jnp.concatenate([sB0[...], sB1[...]], axis=1)
    invA = 1.0 / jnp.maximum(cA[...][:, 0:1], 1.0)
    invB = 1.0 / jnp.maximum(cB[...][:, 0:1], 1.0)
    inter = jnp.dot(sA * invA, wA[...], preferred_element_type=jnp.float32)
    intra = jnp.dot(sB * invB, wB[...], preferred_element_type=jnp.float32)
    f = jnp.concatenate([f0[...], f1[...]], axis=1)
    self_t = jnp.dot(f, wS[...], preferred_element_type=jnp.float32)
    y = jnp.maximum((inter + intra) * 0.5, 0.0) + self_t + bias[...]
    if split:
        outs[0][...] = y[:, :DH]
        outs[1][...] = y[:, DH:]
    else:
        outs[0][...] = y


def _run_dense(sA0, sA1, cA, sB0, sB1, cB, f0, f1, wA, wB, wS, b, split):
    half_spec = pl.BlockSpec((BT, DH), lambda i: (i, 0))
    cnt_spec = pl.BlockSpec((BT, 16), lambda i: (i, 0))
    w_spec = pl.BlockSpec((D, D), lambda i: (0, 0))
    b_spec = pl.BlockSpec((1, D), lambda i: (0, 0))
    if split:
        out_shape = [jax.ShapeDtypeStruct((N, DH), jnp.float32)] * 2
        out_specs = [half_spec, half_spec]
    else:
        out_shape = [jax.ShapeDtypeStruct((N, D), jnp.float32)]
        out_specs = [pl.BlockSpec((BT, D), lambda i: (i, 0))]
    res = pl.pallas_call(
        functools.partial(_dense_block, split),
        grid=(N // BT,),
        in_specs=[half_spec, half_spec, cnt_spec, half_spec, half_spec,
                  cnt_spec, half_spec, half_spec, w_spec, w_spec, w_spec,
                  b_spec],
        out_specs=out_specs,
        out_shape=out_shape,
    )(sA0, sA1, cA, sB0, sB1, cB, f0, f1, wA, wB, wS, b.reshape(1, D))
    return res if split else res[0]


def kernel(feat_l, feat_r, edge_lr, edge_rl, edge_ll, edge_rr,
           W_lr_0, W_ll_0, self_W_0, self_b_0,
           W_lr_1, W_ll_1, self_W_1, self_b_1):
    fl0, fl1 = feat_l[:, :DH], feat_l[:, DH:]
    fr0, fr1 = feat_r[:, :DH], feat_r[:, DH:]

    pad_src = jnp.zeros((EP - E,), jnp.int32)
    pad_dst = jnp.full((EP - E,), TRASH, jnp.int32)

    def pad(e):
        return (jnp.concatenate([e[0], pad_src]).reshape(NB, B),
                jnp.concatenate([e[1], pad_dst]).reshape(NB, B))

    src_lr, dst_lr = pad(edge_lr)
    src_rl, dst_rl = pad(edge_rl)
    src_ll, dst_ll = pad(edge_ll)
    src_rr, dst_rr = pad(edge_rr)
    edges = (src_lr, dst_lr, src_rl, dst_rl, src_ll, dst_ll, src_rr, dst_rr)

    c_lr, c_rl, c_ll, c_rr = _run_counts(dst_lr, dst_rl, dst_ll, dst_rr)

    zeros_h = jnp.zeros((RPT, DH), jnp.float32)

    (s_lr0, s_lr1, s_rl0, s_rl1,
     s_ll0, s_ll1, s_rr0, s_rr1) = _run_agg(fl0, fl1, fr0, fr1, edges,
                                            zeros_h)

    nl0, nl1 = _run_dense(s_rl0, s_rl1, c_rl, s_ll0, s_ll1, c_ll,
                          fl0, fl1, W_lr_0, W_ll_0, self_W_0, self_b_0,
                          split=True)
    nr0, nr1 = _run_dense(s_lr0, s_lr1, c_lr, s_rr0, s_rr1, c_rr,
                          fr0, fr1, W_lr_0, W_ll_0, self_W_0, self_b_0,
                          split=True)

    (t_lr0, t_lr1, t_rl0, t_rl1,
     t_ll0, t_ll1, t_rr0, t_rr1) = _run_agg(nl0, nl1, nr0, nr1, edges,
                                            zeros_h)

    nl = _run_dense(t_rl0, t_rl1, c_rl, t_ll0, t_ll1, c_ll,
                    nl0, nl1, W_lr_1, W_ll_1, self_W_1, self_b_1,
                    split=False)
    nr = _run_dense(t_lr0, t_lr1, c_lr, t_rr0, t_rr1, c_rr,
                    nr0, nr1, W_lr_1, W_ll_1, self_W_1, self_b_1,
                    split=False)

    return jnp.concatenate([nl, nr], axis=0)


# speedup vs baseline: 6.2161x; 1.0286x over previous
import functools

import jax
import jax.numpy as jnp
from jax import lax
from jax.experimental import pallas as pl
from jax.experimental.pallas import tpu as pltpu
from jax.experimental.pallas import tpu_sc as plsc

N = 25000
D = 128
DH = 64
E = 150000
B = 112
NTILES = 16
NBT = 84
NB = NBT * NTILES
EP = NB * B
N_PAD = 25088
RPT = N_PAD // NTILES
TRASH = 25024
BT = 1000


def _agg_body(f0, f1,
              src_lr, dst_lr, src_rl, dst_rl,
              src_ll, dst_ll, src_rr, dst_rr, zeros_h,
              i0, i1, t0, t1,
              spm, sv0, sv1, sv2, sv3, dv0, dv1, dv2, dv3,
              r0, r1, r2, r3,
              gs0, gs1, gs2, gs3, ss0, ss1, ss2, ss3,
              is0, is1, is2, is3, js0, js1, js2, js3):
    c = lax.axis_index("c")
    s = lax.axis_index("s")
    sv = (sv0, sv1, sv2, sv3)
    dv = (dv0, dv1, dv2, dv3)
    rv = (r0, r1, r2, r3)
    gs = (gs0, gs1, gs2, gs3)
    ss = (ss0, ss1, ss2, ss3)
    isem = (is0, is1, is2, is3)
    jsem = (js0, js1, js2, js3)

    def one_etype(feat, src_h, dst_h, out, side):
        row0 = pl.multiple_of(s * RPT, RPT)
        pltpu.sync_copy(zeros_h, spm.at[pl.ds(row0, RPT)])
        plsc.subcore_barrier()
        base = s * NBT
        for j in range(4):
            pltpu.async_copy(src_h.at[base + j], sv[j], isem[j])
        for j in range(2):
            pltpu.async_copy(dst_h.at[base + j], dv[j], jsem[j])
        for j in range(2):
            pltpu.make_async_copy(src_h.at[base], sv[j], isem[j]).wait()
            pltpu.async_copy(feat.at[sv[j]], rv[j], gs[j])

        def step(m, j):
            k = 4 * m + j
            p = j
            n = (j + 2) % 4

            def wait_scatter_km2():
                pltpu.make_async_copy(rv[n], spm.at[dv[n]], ss[n]).wait()

            def fill_n():
                pltpu.async_copy(dst_h.at[base + k + 2], dv[n], jsem[n])
                pltpu.make_async_copy(src_h.at[base], sv[n], isem[n]).wait()
                pltpu.async_copy(feat.at[sv[n]], rv[n], gs[n])

            if j < 2:
                m_gt0 = m > 0
                pl.when(m_gt0)(wait_scatter_km2)
                fill_n()
            else:
                wait_scatter_km2()
                pl.when(4 * m + j + 2 < NBT)(fill_n)
            pltpu.make_async_copy(feat.at[sv[p]], rv[p], gs[p]).wait()
            pltpu.make_async_copy(dst_h.at[base], dv[p], jsem[p]).wait()
            pltpu.async_copy(rv[p], spm.at[dv[p]], ss[p], add=True)

            @pl.when(4 * m + j + 4 < NBT)
            def _():
                pltpu.async_copy(src_h.at[base + k + 4], sv[p], isem[p])

        def body(m, carry):
            for j in range(4):
                step(m, j)
            return carry

        lax.fori_loop(0, NBT // 4, body, 0)
        for j in (2, 3):
            pltpu.make_async_copy(rv[j], spm.at[dv[j]], ss[j]).wait()
        plsc.subcore_barrier()
        pltpu.sync_copy(spm.at[pl.ds(row0, RPT)],
                        out.at[side, pl.ds(row0, RPT)])
        plsc.subcore_barrier()

    def half(feat, inter, intra):
        one_etype(feat, src_lr, dst_lr, inter, 1)
        one_etype(feat, src_rl, dst_rl, inter, 0)
        one_etype(feat, src_ll, dst_ll, intra, 0)
        one_etype(feat, src_rr, dst_rr, intra, 1)

    @pl.when(c == 0)
    def _():
        half(f0, i0, t0)

    @pl.when(c == 1)
    def _():
        half(f1, i1, t1)


def _run_agg(f0, f1, edges, zeros_h):
    mesh = plsc.VectorSubcoreMesh(core_axis_name="c", subcore_axis_name="s")
    out_type = [jax.ShapeDtypeStruct((2, N_PAD, DH), jnp.float32)] * 4
    scratch = (
        [pltpu.VMEM_SHARED((N_PAD, DH), jnp.float32)]
        + [pltpu.VMEM((B,), jnp.int32) for _ in range(8)]
        + [pltpu.VMEM((B, DH), jnp.float32) for _ in range(4)]
        + [pltpu.SemaphoreType.DMA for _ in range(16)]
    )
    fn = pl.kernel(_agg_body, out_type=out_type, mesh=mesh,
                   scratch_types=scratch,
                   compiler_params=pltpu.CompilerParams(
                       use_tc_tiling_on_sc=False))
    return fn(f0, f1, *edges, zeros_h)


def _counts_body(dst_lr, dst_rl, dst_ll, dst_rr, ones_h, zeros_h,
                 c_inter, c_intra,
                 spm, ones_v, zbuf, dv0, dv1, dv2, dv3,
                 ss0, ss1, ss2, ss3, js0, js1, js2, js3):
    c = lax.axis_index("c")
    s = lax.axis_index("s")
    dv = (dv0, dv1, dv2, dv3)
    ss = (ss0, ss1, ss2, ss3)
    jsem = (js0, js1, js2, js3)
    pltpu.sync_copy(ones_h, ones_v)
    pltpu.sync_copy(zeros_h, zbuf)

    def one(dst_h, out, side):
        row0 = pl.multiple_of(s * RPT, RPT)
        pltpu.sync_copy(zbuf, spm.at[pl.ds(row0, RPT)])
        plsc.subcore_barrier()
        base = s * NBT
        for j in range(2):
            pltpu.async_copy(dst_h.at[base + j], dv[j], jsem[j])

        def step(m, j):
            k = 4 * m + j
            p = j
            n = (j + 2) % 4

            def refill():
                pltpu.make_async_copy(ones_v, spm.at[dv[n]], ss[n]).wait()
                pltpu.async_copy(dst_h.at[base + k + 2], dv[n], jsem[n])

            if j < 2:
                pl.when(m > 0)(lambda: pltpu.make_async_copy(
                    ones_v, spm.at[dv[n]], ss[n]).wait())
                pltpu.async_copy(dst_h.at[base + k + 2], dv[n], jsem[n])
            else:
                pl.when(4 * m + j + 2 < NBT)(refill)
                pl.when(4 * m + j + 2 >= NBT)(lambda: pltpu.make_async_copy(
                    ones_v, spm.at[dv[n]], ss[n]).wait())
            pltpu.make_async_copy(dst_h.at[base], dv[p], jsem[p]).wait()
            pltpu.async_copy(ones_v, spm.at[dv[p]], ss[p], add=True)

        def body(m, carry):
            for j in range(4):
                step(m, j)
            return carry

        lax.fori_loop(0, NBT // 4, body, 0)
        for j in (2, 3):
            pltpu.make_async_copy(ones_v, spm.at[dv[j]], ss[j]).wait()
        plsc.subcore_barrier()
        pltpu.sync_copy(spm.at[pl.ds(row0, RPT)],
                        out.at[side, pl.ds(row0, RPT)])
        plsc.subcore_barrier()

    @pl.when(c == 0)
    def _():
        one(dst_rl, c_inter, 0)
        one(dst_ll, c_intra, 0)

    @pl.when(c == 1)
    def _():
        one(dst_lr, c_inter, 1)
        one(dst_rr, c_intra, 1)


def _run_counts(dst_lr, dst_rl, dst_ll, dst_rr):
    mesh = plsc.VectorSubcoreMesh(core_axis_name="c", subcore_axis_name="s")
    out_type = [jax.ShapeDtypeStruct((2, N_PAD, 16), jnp.float32)] * 2
    scratch = (
        [pltpu.VMEM_SHARED((N_PAD, 16), jnp.float32),
         pltpu.VMEM((B, 16), jnp.float32),
         pltpu.VMEM((RPT, 16), jnp.float32)]
        + [pltpu.VMEM((B,), jnp.int32) for _ in range(4)]
        + [pltpu.SemaphoreType.DMA for _ in range(8)]
    )
    ones_h = jnp.ones((B, 16), jnp.float32)
    zeros_h = jnp.zeros((RPT, 16), jnp.float32)
    fn = pl.kernel(_counts_body, out_type=out_type, mesh=mesh,
                   scratch_types=scratch,
                   compiler_params=pltpu.CompilerParams(
                       use_tc_tiling_on_sc=False))
    return fn(dst_lr, dst_rl, dst_ll, dst_rr, ones_h, zeros_h)


def _dense_block(split, sA0, sA1, cA, sB0, sB1, cB, f0, f1,
                 wA, wB, wS, bias, *outs):
    sA = jnp.concatenate([sA0[...][0], sA1[...][0]], axis=1)
    sB = jnp.concatenate([sB0[...][0], sB1[...][0]], axis=1)
    invA = 1.0 / jnp.maximum(cA[...][0, :, 0:1], 1.0)
    invB = 1.0 / jnp.maximum(cB[...][0, :, 0:1], 1.0)
    inter = jnp.dot(sA * invA, wA[...], preferred_element_type=jnp.float32)
    intra = jnp.dot(sB * invB, wB[...], preferred_element_type=jnp.float32)
    f = jnp.concatenate([f0[...], f1[...]], axis=1)
    self_t = jnp.dot(f, wS[...], preferred_element_type=jnp.float32)
    y = jnp.maximum((inter + intra) * 0.5, 0.0) + self_t + bias[...]
    if split:
        outs[0][...] = y[:, :DH]
        outs[1][...] = y[:, DH:]
    else:
        outs[0][...] = y


NBLK = N // BT


def _run_dense(sA0, sA1, cA, sB0, sB1, cB, f0, f1, wA, wB, wS, b, split):
    sum_spec = pl.BlockSpec((1, BT, DH), lambda i: (i // NBLK, i % NBLK, 0))
    cnt_spec = pl.BlockSpec((1, BT, 16), lambda i: (i // NBLK, i % NBLK, 0))
    half_spec = pl.BlockSpec((BT, DH), lambda i: (i, 0))
    w_spec = pl.BlockSpec((D, D), lambda i: (0, 0))
    b_spec = pl.BlockSpec((1, D), lambda i: (0, 0))
    if split:
        out_shape = [jax.ShapeDtypeStruct((2 * N, DH), jnp.float32)] * 2
        out_specs = [half_spec, half_spec]
    else:
        out_shape = [jax.ShapeDtypeStruct((2 * N, D), jnp.float32)]
        out_specs = [pl.BlockSpec((BT, D), lambda i: (i, 0))]
    res = pl.pallas_call(
        functools.partial(_dense_block, split),
        grid=(2 * NBLK,),
        in_specs=[sum_spec, sum_spec, cnt_spec, sum_spec, sum_spec,
                  cnt_spec, half_spec, half_spec, w_spec, w_spec, w_spec,
                  b_spec],
        out_specs=out_specs,
        out_shape=out_shape,
    )(sA0, sA1, cA, sB0, sB1, cB, f0, f1, wA, wB, wS, b.reshape(1, D))
    return res if split else res[0]


def kernel(feat_l, feat_r, edge_lr, edge_rl, edge_ll, edge_rr,
           W_lr_0, W_ll_0, self_W_0, self_b_0,
           W_lr_1, W_ll_1, self_W_1, self_b_1):
    f0 = jnp.concatenate([feat_l[:, :DH], feat_r[:, :DH]], axis=0)
    f1 = jnp.concatenate([feat_l[:, DH:], feat_r[:, DH:]], axis=0)

    pad_src = jnp.zeros((EP - E,), jnp.int32)
    pad_dst = jnp.full((EP - E,), TRASH, jnp.int32)

    def pad(e, off):
        return (jnp.concatenate([e[0] + off, pad_src]).reshape(NB, B),
                jnp.concatenate([e[1], pad_dst]).reshape(NB, B))

    src_lr, dst_lr = pad(edge_lr, 0)
    src_rl, dst_rl = pad(edge_rl, N)
    src_ll, dst_ll = pad(edge_ll, 0)
    src_rr, dst_rr = pad(edge_rr, N)
    edges = (src_lr, dst_lr, src_rl, dst_rl, src_ll, dst_ll, src_rr, dst_rr)

    c_inter, c_intra = _run_counts(dst_lr, dst_rl, dst_ll, dst_rr)

    zeros_h = jnp.zeros((RPT, DH), jnp.float32)

    i0, i1, t0, t1 = _run_agg(f0, f1, edges, zeros_h)
    nf0, nf1 = _run_dense(i0, i1, c_inter, t0, t1, c_intra,
                          f0, f1, W_lr_0, W_ll_0, self_W_0, self_b_0,
                          split=True)
    i0, i1, t0, t1 = _run_agg(nf0, nf1, edges, zeros_h)
    return _run_dense(i0, i1, c_inter, t0, t1, c_intra,
                      nf0, nf1, W_lr_1, W_ll_1, self_W_1, self_b_1,
                      split=False)


# speedup vs baseline: 7.6486x; 1.2304x over previous
import functools

import jax
import jax.numpy as jnp
from jax import lax
from jax.experimental import pallas as pl
from jax.experimental.pallas import tpu as pltpu
from jax.experimental.pallas import tpu_sc as plsc

N = 25000
D = 128
DH = 64
E = 150000
B = 112
NTILES = 16
NBT = 84
NB = NBT * NTILES
EP = NB * B
N_PAD = 25088
RPT = N_PAD // NTILES
TRASH = 25024
BT = 1000


def _agg_body(f0, f1,
              src_lr, dst_lr, src_rl, dst_rl,
              src_ll, dst_ll, src_rr, dst_rr, zeros_h,
              inter, intra,
              spm, sv0, sv1, sv2, sv3, dv0, dv1, dv2, dv3,
              r0, r1, r2, r3,
              gs0, gs1, gs2, gs3, ss0, ss1, ss2, ss3,
              is0, is1, is2, is3, js0, js1, js2, js3):
    c = lax.axis_index("c")
    s = lax.axis_index("s")
    sv = (sv0, sv1, sv2, sv3)
    dv = (dv0, dv1, dv2, dv3)
    rv = (r0, r1, r2, r3)
    gs = (gs0, gs1, gs2, gs3)
    ss = (ss0, ss1, ss2, ss3)
    isem = (is0, is1, is2, is3)
    jsem = (js0, js1, js2, js3)

    def one_etype(feat, src_h, dst_h, out, side, col):
        row0 = pl.multiple_of(s * RPT, RPT)
        cds = pl.ds(col, DH)
        pltpu.sync_copy(zeros_h, spm.at[pl.ds(row0, RPT)])
        plsc.subcore_barrier()
        base = s * NBT

        def src_at(k):
            return src_h.at[pl.ds(pl.multiple_of((base + k) * B, B), B)]

        def dst_at(k):
            return dst_h.at[pl.ds(pl.multiple_of((base + k) * B, B), B)]

        for j in range(4):
            pltpu.async_copy(src_at(j), sv[j], isem[j])
        for j in range(2):
            pltpu.async_copy(dst_at(j), dv[j], jsem[j])
        for j in range(2):
            pltpu.make_async_copy(src_at(0), sv[j], isem[j]).wait()
            pltpu.async_copy(feat.at[sv[j]], rv[j], gs[j])

        def step(m, j):
            k = 4 * m + j
            p = j
            n = (j + 2) % 4

            def wait_scatter_km2():
                pltpu.make_async_copy(rv[n], spm.at[dv[n]], ss[n]).wait()

            def fill_n():
                pltpu.async_copy(dst_at(k + 2), dv[n], jsem[n])
                pltpu.make_async_copy(src_at(0), sv[n], isem[n]).wait()
                pltpu.async_copy(feat.at[sv[n]], rv[n], gs[n])

            if j < 2:
                pl.when(m > 0)(wait_scatter_km2)
                fill_n()
            else:
                wait_scatter_km2()
                pl.when(4 * m + j + 2 < NBT)(fill_n)
            pltpu.make_async_copy(feat.at[sv[p]], rv[p], gs[p]).wait()
            pltpu.make_async_copy(dst_at(0), dv[p], jsem[p]).wait()
            pltpu.async_copy(rv[p], spm.at[dv[p]], ss[p], add=True)

            @pl.when(4 * m + j + 4 < NBT)
            def _():
                pltpu.async_copy(src_at(k + 4), sv[p], isem[p])

        def body(m, carry):
            for j in range(4):
                step(m, j)
            return carry

        lax.fori_loop(0, NBT // 4, body, 0)
        for j in (2, 3):
            pltpu.make_async_copy(rv[j], spm.at[dv[j]], ss[j]).wait()
        plsc.subcore_barrier()
        pltpu.sync_copy(spm.at[pl.ds(row0, RPT)],
                        out.at[side, pl.ds(row0, RPT), cds])
        plsc.subcore_barrier()

    def half(feat, col):
        one_etype(feat, src_lr, dst_lr, inter, 1, col)
        one_etype(feat, src_rl, dst_rl, inter, 0, col)
        one_etype(feat, src_ll, dst_ll, intra, 0, col)
        one_etype(feat, src_rr, dst_rr, intra, 1, col)

    @pl.when(c == 0)
    def _():
        half(f0, 0)

    @pl.when(c == 1)
    def _():
        half(f1, DH)


def _run_agg(f0, f1, edges, zeros_h):
    mesh = plsc.VectorSubcoreMesh(core_axis_name="c", subcore_axis_name="s")
    out_type = [jax.ShapeDtypeStruct((2, N_PAD, D), jnp.float32)] * 2
    scratch = (
        [pltpu.VMEM_SHARED((N_PAD, DH), jnp.float32)]
        + [pltpu.VMEM((B,), jnp.int32) for _ in range(8)]
        + [pltpu.VMEM((B, DH), jnp.float32) for _ in range(4)]
        + [pltpu.SemaphoreType.DMA for _ in range(16)]
    )
    fn = pl.kernel(_agg_body, out_type=out_type, mesh=mesh,
                   scratch_types=scratch,
                   compiler_params=pltpu.CompilerParams(
                       use_tc_tiling_on_sc=False))
    return fn(f0, f1, *edges, zeros_h)


def _counts_body(dst_lr, dst_rl, dst_ll, dst_rr, ones_h, zeros_h,
                 c_inter, c_intra,
                 spm, ones_v, zbuf, dv0, dv1, dv2, dv3,
                 ss0, ss1, ss2, ss3, js0, js1, js2, js3):
    c = lax.axis_index("c")
    s = lax.axis_index("s")
    dv = (dv0, dv1, dv2, dv3)
    ss = (ss0, ss1, ss2, ss3)
    jsem = (js0, js1, js2, js3)
    pltpu.sync_copy(ones_h, ones_v)
    pltpu.sync_copy(zeros_h, zbuf)

    def one(dst_h, out, side):
        row0 = pl.multiple_of(s * RPT, RPT)
        pltpu.sync_copy(zbuf, spm.at[pl.ds(row0, RPT)])
        plsc.subcore_barrier()
        base = s * NBT

        def dst_at(k):
            return dst_h.at[pl.ds(pl.multiple_of((base + k) * B, B), B)]

        for j in range(2):
            pltpu.async_copy(dst_at(j), dv[j], jsem[j])

        def step(m, j):
            k = 4 * m + j
            p = j
            n = (j + 2) % 4

            def refill():
                pltpu.make_async_copy(ones_v, spm.at[dv[n]], ss[n]).wait()
                pltpu.async_copy(dst_at(k + 2), dv[n], jsem[n])

            if j < 2:
                pl.when(m > 0)(lambda: pltpu.make_async_copy(
                    ones_v, spm.at[dv[n]], ss[n]).wait())
                pltpu.async_copy(dst_at(k + 2), dv[n], jsem[n])
            else:
                pl.when(4 * m + j + 2 < NBT)(refill)
                pl.when(4 * m + j + 2 >= NBT)(lambda: pltpu.make_async_copy(
                    ones_v, spm.at[dv[n]], ss[n]).wait())
            pltpu.make_async_copy(dst_at(0), dv[p], jsem[p]).wait()
            pltpu.async_copy(ones_v, spm.at[dv[p]], ss[p], add=True)

        def body(m, carry):
            for j in range(4):
                step(m, j)
            return carry

        lax.fori_loop(0, NBT // 4, body, 0)
        for j in (2, 3):
            pltpu.make_async_copy(ones_v, spm.at[dv[j]], ss[j]).wait()
        plsc.subcore_barrier()
        pltpu.sync_copy(spm.at[pl.ds(row0, RPT)],
                        out.at[side, pl.ds(row0, RPT), pl.ds(0, 16)])
        plsc.subcore_barrier()

    @pl.when(c == 0)
    def _():
        one(dst_rl, c_inter, 0)
        one(dst_ll, c_intra, 0)

    @pl.when(c == 1)
    def _():
        one(dst_lr, c_inter, 1)
        one(dst_rr, c_intra, 1)


def _run_counts(dst_lr, dst_rl, dst_ll, dst_rr):
    mesh = plsc.VectorSubcoreMesh(core_axis_name="c", subcore_axis_name="s")
    out_type = [jax.ShapeDtypeStruct((2, N_PAD, D), jnp.float32)] * 2
    scratch = (
        [pltpu.VMEM_SHARED((N_PAD, 16), jnp.float32),
         pltpu.VMEM((B, 16), jnp.float32),
         pltpu.VMEM((RPT, 16), jnp.float32)]
        + [pltpu.VMEM((B,), jnp.int32) for _ in range(4)]
        + [pltpu.SemaphoreType.DMA for _ in range(8)]
    )
    ones_h = jnp.ones((B, 16), jnp.float32)
    zeros_h = jnp.zeros((RPT, 16), jnp.float32)
    fn = pl.kernel(_counts_body, out_type=out_type, mesh=mesh,
                   scratch_types=scratch,
                   compiler_params=pltpu.CompilerParams(
                       use_tc_tiling_on_sc=False))
    return fn(dst_lr, dst_rl, dst_ll, dst_rr, ones_h, zeros_h)


def _dense_block(split, sA, cA, sB, cB, f0, f1, wA, wB, wS, bias, *outs):
    invA = 1.0 / jnp.maximum(cA[...][0, :, 0:1], 1.0)
    invB = 1.0 / jnp.maximum(cB[...][0, :, 0:1], 1.0)
    inter = jnp.dot(sA[...][0] * invA, wA[...],
                    preferred_element_type=jnp.float32)
    intra = jnp.dot(sB[...][0] * invB, wB[...],
                    preferred_element_type=jnp.float32)
    f = jnp.concatenate([f0[...], f1[...]], axis=1)
    self_t = jnp.dot(f, wS[...], preferred_element_type=jnp.float32)
    y = jnp.maximum((inter + intra) * 0.5, 0.0) + self_t + bias[...]
    if split:
        outs[0][...] = y[:, :DH]
        outs[1][...] = y[:, DH:]
    else:
        outs[0][...] = y


NBLK = N // BT


def _run_dense(sA, cA, sB, cB, f0, f1, wA, wB, wS, b, split):
    sum_spec = pl.BlockSpec((1, BT, D), lambda i: (i // NBLK, i % NBLK, 0))
    cnt_spec = pl.BlockSpec((1, BT, D), lambda i: (i // NBLK, i % NBLK, 0))
    half_spec = pl.BlockSpec((BT, DH), lambda i: (i, 0))
    w_spec = pl.BlockSpec((D, D), lambda i: (0, 0))
    b_spec = pl.BlockSpec((1, D), lambda i: (0, 0))
    if split:
        out_shape = [jax.ShapeDtypeStruct((2 * N, DH), jnp.float32)] * 2
        out_specs = [half_spec, half_spec]
    else:
        out_shape = [jax.ShapeDtypeStruct((2 * N, D), jnp.float32)]
        out_specs = [pl.BlockSpec((BT, D), lambda i: (i, 0))]
    res = pl.pallas_call(
        functools.partial(_dense_block, split),
        grid=(2 * NBLK,),
        in_specs=[sum_spec, cnt_spec, sum_spec, cnt_spec, half_spec,
                  half_spec, w_spec, w_spec, w_spec, b_spec],
        out_specs=out_specs,
        out_shape=out_shape,
    )(sA, cA, sB, cB, f0, f1, wA, wB, wS, b.reshape(1, D))
    return res if split else res[0]


def kernel(feat_l, feat_r, edge_lr, edge_rl, edge_ll, edge_rr,
           W_lr_0, W_ll_0, self_W_0, self_b_0,
           W_lr_1, W_ll_1, self_W_1, self_b_1):
    f0 = jnp.concatenate([feat_l[:, :DH], feat_r[:, :DH]], axis=0)
    f1 = jnp.concatenate([feat_l[:, DH:], feat_r[:, DH:]], axis=0)

    pad_src = jnp.zeros((EP - E,), jnp.int32)
    pad_dst = jnp.full((EP - E,), TRASH, jnp.int32)

    def pad(e, off):
        return (jnp.concatenate([e[0] + off, pad_src]),
                jnp.concatenate([e[1], pad_dst]))

    src_lr, dst_lr = pad(edge_lr, 0)
    src_rl, dst_rl = pad(edge_rl, N)
    src_ll, dst_ll = pad(edge_ll, 0)
    src_rr, dst_rr = pad(edge_rr, N)
    edges = (src_lr, dst_lr, src_rl, dst_rl, src_ll, dst_ll, src_rr, dst_rr)

    c_inter, c_intra = _run_counts(dst_lr, dst_rl, dst_ll, dst_rr)

    zeros_h = jnp.zeros((RPT, DH), jnp.float32)

    inter, intra = _run_agg(f0, f1, edges, zeros_h)
    nf0, nf1 = _run_dense(inter, c_inter, intra, c_intra, f0, f1,
                          W_lr_0, W_ll_0, self_W_0, self_b_0, split=True)
    inter, intra = _run_agg(nf0, nf1, edges, zeros_h)
    return _run_dense(inter, c_inter, intra, c_intra, nf0, nf1,
                      W_lr_1, W_ll_1, self_W_1, self_b_1, split=False)


# speedup vs baseline: 7.8102x; 1.0211x over previous
import functools

import jax
import jax.numpy as jnp
from jax import lax
from jax.experimental import pallas as pl
from jax.experimental.pallas import tpu as pltpu
from jax.experimental.pallas import tpu_sc as plsc

N = 25000
D = 128
DH = 64
E = 150000
B = 112
NTILES = 16
NBT = 84
NB = NBT * NTILES
EP = NB * B
N_PAD = 25088
RPT = N_PAD // NTILES
TRASH = 25024
BT = 1000


def _agg_body(f0, f1,
              src_lr, dst_lr, src_rl, dst_rl,
              src_ll, dst_ll, src_rr, dst_rr, zeros_h,
              inter, intra,
              spm, sv0, sv1, sv2, sv3, dv0, dv1, dv2, dv3,
              r0, r1, r2, r3,
              gs0, gs1, gs2, gs3, ss0, ss1, ss2, ss3,
              is0, is1, is2, is3, js0, js1, js2, js3):
    c = lax.axis_index("c")
    s = lax.axis_index("s")
    sv = (sv0, sv1, sv2, sv3)
    dv = (dv0, dv1, dv2, dv3)
    rv = (r0, r1, r2, r3)
    gs = (gs0, gs1, gs2, gs3)
    ss = (ss0, ss1, ss2, ss3)
    isem = (is0, is1, is2, is3)
    jsem = (js0, js1, js2, js3)

    row0 = pl.multiple_of(s * RPT, RPT)
    base = s * NBT

    def _src_at(src_h, k):
        return src_h.at[pl.ds(pl.multiple_of((base + k) * B, B), B)]

    def prologue(feat, src_h, dst_h):
        for j in range(4):
            pltpu.async_copy(_src_at(src_h, j), sv[j], isem[j])
        for j in range(2):
            pltpu.async_copy(_src_at(dst_h, j), dv[j], jsem[j])
        for j in range(2):
            pltpu.make_async_copy(_src_at(src_h, 0), sv[j], isem[j]).wait()
            pltpu.async_copy(feat.at[sv[j]], rv[j], gs[j])

    def copyout(out, side, col):
        pltpu.sync_copy(spm.at[pl.ds(row0, RPT)],
                        out.at[side, pl.ds(row0, RPT), pl.ds(col, DH)])

    def one_etype(feat, src_h, dst_h):
        src_at = lambda k: _src_at(src_h, k)
        dst_at = lambda k: _src_at(dst_h, k)

        def step(m, j):
            k = 4 * m + j
            p = j
            n = (j + 2) % 4

            def wait_scatter_km2():
                pltpu.make_async_copy(rv[n], spm.at[dv[n]], ss[n]).wait()

            def fill_n():
                pltpu.async_copy(dst_at(k + 2), dv[n], jsem[n])
                pltpu.make_async_copy(src_at(0), sv[n], isem[n]).wait()
                pltpu.async_copy(feat.at[sv[n]], rv[n], gs[n])

            if j < 2:
                pl.when(m > 0)(wait_scatter_km2)
                fill_n()
            else:
                wait_scatter_km2()
                pl.when(4 * m + j + 2 < NBT)(fill_n)
            pltpu.make_async_copy(feat.at[sv[p]], rv[p], gs[p]).wait()
            pltpu.make_async_copy(dst_at(0), dv[p], jsem[p]).wait()
            pltpu.async_copy(rv[p], spm.at[dv[p]], ss[p], add=True)

            @pl.when(4 * m + j + 4 < NBT)
            def _():
                pltpu.async_copy(src_at(k + 4), sv[p], isem[p])

        def body(m, carry):
            for j in range(4):
                step(m, j)
            return carry

        lax.fori_loop(0, NBT // 4, body, 0)
        for j in (2, 3):
            pltpu.make_async_copy(rv[j], spm.at[dv[j]], ss[j]).wait()
        plsc.subcore_barrier()

    def half(feat, col):
        ets = ((src_lr, dst_lr, inter, 1), (src_rl, dst_rl, inter, 0),
               (src_ll, dst_ll, intra, 0), (src_rr, dst_rr, intra, 1))
        for i, (src_h, dst_h, out, side) in enumerate(ets):
            prologue(feat, src_h, dst_h)
            if i > 0:
                copyout(ets[i - 1][2], ets[i - 1][3], col)
            pltpu.sync_copy(zeros_h, spm.at[pl.ds(row0, RPT)])
            plsc.subcore_barrier()
            one_etype(feat, src_h, dst_h)
        copyout(ets[3][2], ets[3][3], col)

    @pl.when(c == 0)
    def _():
        half(f0, 0)

    @pl.when(c == 1)
    def _():
        half(f1, DH)


def _run_agg(f0, f1, edges, zeros_h):
    mesh = plsc.VectorSubcoreMesh(core_axis_name="c", subcore_axis_name="s")
    out_type = [jax.ShapeDtypeStruct((2, N_PAD, D), jnp.float32)] * 2
    scratch = (
        [pltpu.VMEM_SHARED((N_PAD, DH), jnp.float32)]
        + [pltpu.VMEM((B,), jnp.int32) for _ in range(8)]
        + [pltpu.VMEM((B, DH), jnp.float32) for _ in range(4)]
        + [pltpu.SemaphoreType.DMA for _ in range(16)]
    )
    fn = pl.kernel(_agg_body, out_type=out_type, mesh=mesh,
                   scratch_types=scratch,
                   compiler_params=pltpu.CompilerParams(
                       use_tc_tiling_on_sc=False))
    return fn(f0, f1, *edges, zeros_h)


def _counts_body(dst_lr, dst_rl, dst_ll, dst_rr, ones_h, zeros_h,
                 c_inter, c_intra,
                 spm, ones_v, zbuf, dv0, dv1, dv2, dv3,
                 ss0, ss1, ss2, ss3, js0, js1, js2, js3):
    c = lax.axis_index("c")
    s = lax.axis_index("s")
    dv = (dv0, dv1, dv2, dv3)
    ss = (ss0, ss1, ss2, ss3)
    jsem = (js0, js1, js2, js3)
    pltpu.sync_copy(ones_h, ones_v)
    pltpu.sync_copy(zeros_h, zbuf)

    def one(dst_h, out, side):
        row0 = pl.multiple_of(s * RPT, RPT)
        pltpu.sync_copy(zbuf, spm.at[pl.ds(row0, RPT)])
        plsc.subcore_barrier()
        base = s * NBT

        def dst_at(k):
            return dst_h.at[pl.ds(pl.multiple_of((base + k) * B, B), B)]

        for j in range(2):
            pltpu.async_copy(dst_at(j), dv[j], jsem[j])

        def step(m, j):
            k = 4 * m + j
            p = j
            n = (j + 2) % 4

            def refill():
                pltpu.make_async_copy(ones_v, spm.at[dv[n]], ss[n]).wait()
                pltpu.async_copy(dst_at(k + 2), dv[n], jsem[n])

            if j < 2:
                pl.when(m > 0)(lambda: pltpu.make_async_copy(
                    ones_v, spm.at[dv[n]], ss[n]).wait())
                pltpu.async_copy(dst_at(k + 2), dv[n], jsem[n])
            else:
                pl.when(4 * m + j + 2 < NBT)(refill)
                pl.when(4 * m + j + 2 >= NBT)(lambda: pltpu.make_async_copy(
                    ones_v, spm.at[dv[n]], ss[n]).wait())
            pltpu.make_async_copy(dst_at(0), dv[p], jsem[p]).wait()
            pltpu.async_copy(ones_v, spm.at[dv[p]], ss[p], add=True)

        def body(m, carry):
            for j in range(4):
                step(m, j)
            return carry

        lax.fori_loop(0, NBT // 4, body, 0)
        for j in (2, 3):
            pltpu.make_async_copy(ones_v, spm.at[dv[j]], ss[j]).wait()
        plsc.subcore_barrier()
        pltpu.sync_copy(spm.at[pl.ds(row0, RPT)],
                        out.at[side, pl.ds(row0, RPT), pl.ds(0, 16)])
        plsc.subcore_barrier()

    @pl.when(c == 0)
    def _():
        one(dst_rl, c_inter, 0)
        one(dst_ll, c_intra, 0)

    @pl.when(c == 1)
    def _():
        one(dst_lr, c_inter, 1)
        one(dst_rr, c_intra, 1)


def _run_counts(dst_lr, dst_rl, dst_ll, dst_rr):
    mesh = plsc.VectorSubcoreMesh(core_axis_name="c", subcore_axis_name="s")
    out_type = [jax.ShapeDtypeStruct((2, N_PAD, D), jnp.float32)] * 2
    scratch = (
        [pltpu.VMEM_SHARED((N_PAD, 16), jnp.float32),
         pltpu.VMEM((B, 16), jnp.float32),
         pltpu.VMEM((RPT, 16), jnp.float32)]
        + [pltpu.VMEM((B,), jnp.int32) for _ in range(4)]
        + [pltpu.SemaphoreType.DMA for _ in range(8)]
    )
    ones_h = jnp.ones((B, 16), jnp.float32)
    zeros_h = jnp.zeros((RPT, 16), jnp.float32)
    fn = pl.kernel(_counts_body, out_type=out_type, mesh=mesh,
                   scratch_types=scratch,
                   compiler_params=pltpu.CompilerParams(
                       use_tc_tiling_on_sc=False))
    return fn(dst_lr, dst_rl, dst_ll, dst_rr, ones_h, zeros_h)


def _dense_block(split, sA, cA, sB, cB, f0, f1, wAB, wS, bias, *outs):
    invA = 1.0 / jnp.maximum(cA[...][0, :, 0:1], 1.0)
    invB = 1.0 / jnp.maximum(cB[...][0, :, 0:1], 1.0)
    sAB = jnp.concatenate([sA[...][0] * invA, sB[...][0] * invB], axis=1)
    mm = jnp.dot(sAB, wAB[...], preferred_element_type=jnp.float32)
    f = jnp.concatenate([f0[...], f1[...]], axis=1)
    self_t = jnp.dot(f, wS[...], preferred_element_type=jnp.float32)
    y = jnp.maximum(mm * 0.5, 0.0) + self_t + bias[...]
    if split:
        outs[0][...] = y[:, :DH]
        outs[1][...] = y[:, DH:]
    else:
        outs[0][...] = y


NBLK = N // BT


def _edge_prep_block(e_lr, e_rl, e_ll, e_rr, *outs):
    zpad = jnp.zeros((EP - E,), jnp.int32)
    tpad = jnp.full((EP - E,), TRASH, jnp.int32)
    for i, (e, off) in enumerate(((e_lr, 0), (e_rl, N),
                                  (e_ll, 0), (e_rr, N))):
        outs[2 * i][pl.ds(0, E)] = e[0, :] + off
        outs[2 * i][pl.ds(E, EP - E)] = zpad
        outs[2 * i + 1][pl.ds(0, E)] = e[1, :]
        outs[2 * i + 1][pl.ds(E, EP - E)] = tpad


def _edge_prep(e_lr, e_rl, e_ll, e_rr):
    return pl.pallas_call(
        _edge_prep_block,
        out_shape=[jax.ShapeDtypeStruct((EP,), jnp.int32)] * 8,
    )(e_lr, e_rl, e_ll, e_rr)


def _run_dense(sA, cA, sB, cB, f0, f1, wA, wB, wS, b, split):
    sum_spec = pl.BlockSpec((1, BT, D), lambda i: (i // NBLK, i % NBLK, 0))
    cnt_spec = pl.BlockSpec((1, BT, D), lambda i: (i // NBLK, i % NBLK, 0))
    half_spec = pl.BlockSpec((BT, DH), lambda i: (i, 0))
    wab_spec = pl.BlockSpec((2 * D, D), lambda i: (0, 0))
    w_spec = pl.BlockSpec((D, D), lambda i: (0, 0))
    b_spec = pl.BlockSpec((1, D), lambda i: (0, 0))
    if split:
        out_shape = [jax.ShapeDtypeStruct((2 * N, DH), jnp.float32)] * 2
        out_specs = [half_spec, half_spec]
    else:
        out_shape = [jax.ShapeDtypeStruct((2 * N, D), jnp.float32)]
        out_specs = [pl.BlockSpec((BT, D), lambda i: (i, 0))]
    wab = jnp.concatenate([wA, wB], axis=0)
    res = pl.pallas_call(
        functools.partial(_dense_block, split),
        grid=(2 * NBLK,),
        in_specs=[sum_spec, cnt_spec, sum_spec, cnt_spec, half_spec,
                  half_spec, wab_spec, w_spec, b_spec],
        out_specs=out_specs,
        out_shape=out_shape,
    )(sA, cA, sB, cB, f0, f1, wab, wS, b.reshape(1, D))
    return res if split else res[0]


def kernel(feat_l, feat_r, edge_lr, edge_rl, edge_ll, edge_rr,
           W_lr_0, W_ll_0, self_W_0, self_b_0,
           W_lr_1, W_ll_1, self_W_1, self_b_1):
    f0 = jnp.concatenate([feat_l[:, :DH], feat_r[:, :DH]], axis=0)
    f1 = jnp.concatenate([feat_l[:, DH:], feat_r[:, DH:]], axis=0)

    edges = _edge_prep(edge_lr, edge_rl, edge_ll, edge_rr)
    (src_lr, dst_lr, src_rl, dst_rl,
     src_ll, dst_ll, src_rr, dst_rr) = edges

    c_inter, c_intra = _run_counts(dst_lr, dst_rl, dst_ll, dst_rr)

    zeros_h = jnp.zeros((RPT, DH), jnp.float32)

    inter, intra = _run_agg(f0, f1, edges, zeros_h)
    nf0, nf1 = _run_dense(inter, c_inter, intra, c_intra, f0, f1,
                          W_lr_0, W_ll_0, self_W_0, self_b_0, split=True)
    inter, intra = _run_agg(nf0, nf1, edges, zeros_h)
    return _run_dense(inter, c_inter, intra, c_intra, nf0, nf1,
                      W_lr_1, W_ll_1, self_W_1, self_b_1, split=False)


# speedup vs baseline: 8.7479x; 1.1201x over previous
import functools

import jax
import jax.numpy as jnp
from jax import lax
from jax.experimental import pallas as pl
from jax.experimental.pallas import tpu as pltpu
from jax.experimental.pallas import tpu_sc as plsc

N = 25000
D = 128
DH = 64
E = 150000
B = 112
NTILES = 16
NBT = 84
NB = NBT * NTILES
EP = NB * B
N_PAD = 25088
RPT = N_PAD // NTILES
TRASH = 25024
BT = 1000


def _agg_body(ftab,
              sa_lr, sb_lr, dst_lr, sa_rl, sb_rl, dst_rl,
              sa_ll, sb_ll, dst_ll, sa_rr, sb_rr, dst_rr, zeros_h,
              inter, intra,
              spm, sv0, sv1, sv2, sv3, dv0, dv1, dv2, dv3,
              r0, r1, r2, r3,
              gs0, gs1, gs2, gs3, ss0, ss1, ss2, ss3,
              is0, is1, is2, is3, js0, js1, js2, js3):
    c = lax.axis_index("c")
    s = lax.axis_index("s")
    sv = (sv0, sv1, sv2, sv3)
    dv = (dv0, dv1, dv2, dv3)
    rv = (r0, r1, r2, r3)
    gs = (gs0, gs1, gs2, gs3)
    ss = (ss0, ss1, ss2, ss3)
    isem = (is0, is1, is2, is3)
    jsem = (js0, js1, js2, js3)

    row0 = pl.multiple_of(s * RPT, RPT)
    base = s * NBT

    def _src_at(src_h, k):
        return src_h.at[pl.ds(pl.multiple_of((base + k) * B, B), B)]

    def prologue(src_h, dst_h):
        for j in range(4):
            pltpu.async_copy(_src_at(src_h, j), sv[j], isem[j])
        for j in range(2):
            pltpu.async_copy(_src_at(dst_h, j), dv[j], jsem[j])
        for j in range(2):
            pltpu.make_async_copy(_src_at(src_h, 0), sv[j], isem[j]).wait()
            pltpu.async_copy(ftab.at[sv[j]], rv[j], gs[j])

    def copyout(out, side, col):
        pltpu.sync_copy(spm.at[pl.ds(row0, RPT)],
                        out.at[side, pl.ds(row0, RPT), pl.ds(col, DH)])

    def one_etype(src_h, dst_h):
        src_at = lambda k: _src_at(src_h, k)
        dst_at = lambda k: _src_at(dst_h, k)

        def step(m, j):
            k = 4 * m + j
            p = j
            n = (j + 2) % 4

            def wait_scatter_km2():
                pltpu.make_async_copy(rv[n], spm.at[dv[n]], ss[n]).wait()

            def fill_n():
                pltpu.async_copy(dst_at(k + 2), dv[n], jsem[n])
                pltpu.make_async_copy(src_at(0), sv[n], isem[n]).wait()
                pltpu.async_copy(ftab.at[sv[n]], rv[n], gs[n])

            if j < 2:
                pl.when(m > 0)(wait_scatter_km2)
                fill_n()
            else:
                wait_scatter_km2()
                pl.when(4 * m + j + 2 < NBT)(fill_n)
            pltpu.make_async_copy(ftab.at[sv[p]], rv[p], gs[p]).wait()
            pltpu.make_async_copy(dst_at(0), dv[p], jsem[p]).wait()
            pltpu.async_copy(rv[p], spm.at[dv[p]], ss[p], add=True)

            @pl.when(4 * m + j + 4 < NBT)
            def _():
                pltpu.async_copy(src_at(k + 4), sv[p], isem[p])

        def body(m, carry):
            for j in range(4):
                step(m, j)
            return carry

        lax.fori_loop(0, NBT // 4, body, 0)
        for j in (2, 3):
            pltpu.make_async_copy(rv[j], spm.at[dv[j]], ss[j]).wait()
        plsc.subcore_barrier()

    def half(ets, col):
        for i, (src_h, dst_h, out, side) in enumerate(ets):
            prologue(src_h, dst_h)
            if i > 0:
                copyout(ets[i - 1][2], ets[i - 1][3], col)
            pltpu.sync_copy(zeros_h, spm.at[pl.ds(row0, RPT)])
            plsc.subcore_barrier()
            one_etype(src_h, dst_h)
        copyout(ets[3][2], ets[3][3], col)

    @pl.when(c == 0)
    def _():
        half(((sa_lr, dst_lr, inter, 1), (sa_rl, dst_rl, inter, 0),
              (sa_ll, dst_ll, intra, 0), (sa_rr, dst_rr, intra, 1)), 0)

    @pl.when(c == 1)
    def _():
        half(((sb_lr, dst_lr, inter, 1), (sb_rl, dst_rl, inter, 0),
              (sb_ll, dst_ll, intra, 0), (sb_rr, dst_rr, intra, 1)), DH)


def _run_agg(ftab, edges, zeros_h):
    mesh = plsc.VectorSubcoreMesh(core_axis_name="c", subcore_axis_name="s")
    out_type = [jax.ShapeDtypeStruct((2, N_PAD, D), jnp.float32)] * 2
    scratch = (
        [pltpu.VMEM_SHARED((N_PAD, DH), jnp.float32)]
        + [pltpu.VMEM((B,), jnp.int32) for _ in range(8)]
        + [pltpu.VMEM((B, DH), jnp.float32) for _ in range(4)]
        + [pltpu.SemaphoreType.DMA for _ in range(16)]
    )
    fn = pl.kernel(_agg_body, out_type=out_type, mesh=mesh,
                   scratch_types=scratch,
                   compiler_params=pltpu.CompilerParams(
                       use_tc_tiling_on_sc=False))
    return fn(ftab, *edges, zeros_h)


def _counts_body(dst_lr, dst_rl, dst_ll, dst_rr, ones_h, zeros_h,
                 c_inter, c_intra,
                 spm, ones_v, zbuf, dv0, dv1, dv2, dv3,
                 ss0, ss1, ss2, ss3, js0, js1, js2, js3):
    c = lax.axis_index("c")
    s = lax.axis_index("s")
    dv = (dv0, dv1, dv2, dv3)
    ss = (ss0, ss1, ss2, ss3)
    jsem = (js0, js1, js2, js3)
    pltpu.sync_copy(ones_h, ones_v)
    pltpu.sync_copy(zeros_h, zbuf)

    def one(dst_h, out, side):
        row0 = pl.multiple_of(s * RPT, RPT)
        pltpu.sync_copy(zbuf, spm.at[pl.ds(row0, RPT)])
        plsc.subcore_barrier()
        base = s * NBT

        def dst_at(k):
            return dst_h.at[pl.ds(pl.multiple_of((base + k) * B, B), B)]

        for j in range(2):
            pltpu.async_copy(dst_at(j), dv[j], jsem[j])

        def step(m, j):
            k = 4 * m + j
            p = j
            n = (j + 2) % 4

            def refill():
                pltpu.make_async_copy(ones_v, spm.at[dv[n]], ss[n]).wait()
                pltpu.async_copy(dst_at(k + 2), dv[n], jsem[n])

            if j < 2:
                pl.when(m > 0)(lambda: pltpu.make_async_copy(
                    ones_v, spm.at[dv[n]], ss[n]).wait())
                pltpu.async_copy(dst_at(k + 2), dv[n], jsem[n])
            else:
                pl.when(4 * m + j + 2 < NBT)(refill)
                pl.when(4 * m + j + 2 >= NBT)(lambda: pltpu.make_async_copy(
                    ones_v, spm.at[dv[n]], ss[n]).wait())
            pltpu.make_async_copy(dst_at(0), dv[p], jsem[p]).wait()
            pltpu.async_copy(ones_v, spm.at[dv[p]], ss[p], add=True)

        def body(m, carry):
            for j in range(4):
                step(m, j)
            return carry

        lax.fori_loop(0, NBT // 4, body, 0)
        for j in (2, 3):
            pltpu.make_async_copy(ones_v, spm.at[dv[j]], ss[j]).wait()
        plsc.subcore_barrier()
        pltpu.sync_copy(spm.at[pl.ds(row0, RPT)],
                        out.at[side, pl.ds(row0, RPT), pl.ds(0, 16)])
        plsc.subcore_barrier()

    @pl.when(c == 0)
    def _():
        one(dst_rl, c_inter, 0)
        one(dst_ll, c_intra, 0)

    @pl.when(c == 1)
    def _():
        one(dst_lr, c_inter, 1)
        one(dst_rr, c_intra, 1)


def _run_counts(dst_lr, dst_rl, dst_ll, dst_rr):
    mesh = plsc.VectorSubcoreMesh(core_axis_name="c", subcore_axis_name="s")
    out_type = [jax.ShapeDtypeStruct((2, N_PAD, D), jnp.float32)] * 2
    scratch = (
        [pltpu.VMEM_SHARED((N_PAD, 16), jnp.float32),
         pltpu.VMEM((B, 16), jnp.float32),
         pltpu.VMEM((RPT, 16), jnp.float32)]
        + [pltpu.VMEM((B,), jnp.int32) for _ in range(4)]
        + [pltpu.SemaphoreType.DMA for _ in range(8)]
    )
    ones_h = jnp.ones((B, 16), jnp.float32)
    zeros_h = jnp.zeros((RPT, 16), jnp.float32)
    fn = pl.kernel(_counts_body, out_type=out_type, mesh=mesh,
                   scratch_types=scratch,
                   compiler_params=pltpu.CompilerParams(
                       use_tc_tiling_on_sc=False))
    return fn(dst_lr, dst_rl, dst_ll, dst_rr, ones_h, zeros_h)


def _dense_block(sA, cA, sB, cB, f, wAB, wS, bias, out):
    invA = 1.0 / jnp.maximum(cA[...][0, :, 0:1], 1.0)
    invB = 1.0 / jnp.maximum(cB[...][0, :, 0:1], 1.0)
    sAB = jnp.concatenate([sA[...][0] * invA, sB[...][0] * invB], axis=1)
    mm = jnp.dot(sAB, wAB[...], preferred_element_type=jnp.float32)
    self_t = jnp.dot(f[...], wS[...], preferred_element_type=jnp.float32)
    out[...] = jnp.maximum(mm * 0.5, 0.0) + self_t + bias[...]


NBLK = N // BT


def _edge_prep_block(e_lr, e_rl, e_ll, e_rr, *outs):
    zpad = jnp.zeros((EP - E,), jnp.int32)
    tpad = jnp.full((EP - E,), TRASH, jnp.int32)
    for i, (e, off) in enumerate(((e_lr, 0), (e_rl, N),
                                  (e_ll, 0), (e_rr, N))):
        s2 = 2 * (e[0, :] + off)
        outs[3 * i][pl.ds(0, E)] = s2
        outs[3 * i][pl.ds(E, EP - E)] = zpad
        outs[3 * i + 1][pl.ds(0, E)] = s2 + 1
        outs[3 * i + 1][pl.ds(E, EP - E)] = zpad
        outs[3 * i + 2][pl.ds(0, E)] = e[1, :]
        outs[3 * i + 2][pl.ds(E, EP - E)] = tpad


def _edge_prep(e_lr, e_rl, e_ll, e_rr):
    return pl.pallas_call(
        _edge_prep_block,
        out_shape=[jax.ShapeDtypeStruct((EP,), jnp.int32)] * 12,
    )(e_lr, e_rl, e_ll, e_rr)


def _run_dense(sA, cA, sB, cB, f, wA, wB, wS, b):
    sum_spec = pl.BlockSpec((1, BT, D), lambda i: (i // NBLK, i % NBLK, 0))
    cnt_spec = pl.BlockSpec((1, BT, D), lambda i: (i // NBLK, i % NBLK, 0))
    f_spec = pl.BlockSpec((BT, D), lambda i: (i, 0))
    wab_spec = pl.BlockSpec((2 * D, D), lambda i: (0, 0))
    w_spec = pl.BlockSpec((D, D), lambda i: (0, 0))
    b_spec = pl.BlockSpec((1, D), lambda i: (0, 0))
    wab = jnp.concatenate([wA, wB], axis=0)
    return pl.pallas_call(
        _dense_block,
        grid=(2 * NBLK,),
        in_specs=[sum_spec, cnt_spec, sum_spec, cnt_spec, f_spec,
                  wab_spec, w_spec, b_spec],
        out_specs=f_spec,
        out_shape=jax.ShapeDtypeStruct((2 * N, D), jnp.float32),
    )(sA, cA, sB, cB, f, wab, wS, b.reshape(1, D))


def kernel(feat_l, feat_r, edge_lr, edge_rl, edge_ll, edge_rr,
           W_lr_0, W_ll_0, self_W_0, self_b_0,
           W_lr_1, W_ll_1, self_W_1, self_b_1):
    feat = jnp.concatenate([feat_l, feat_r], axis=0)

    edges = _edge_prep(edge_lr, edge_rl, edge_ll, edge_rr)
    dst_lr, dst_rl, dst_ll, dst_rr = edges[2], edges[5], edges[8], edges[11]

    c_inter, c_intra = _run_counts(dst_lr, dst_rl, dst_ll, dst_rr)

    zeros_h = jnp.zeros((RPT, DH), jnp.float32)

    inter, intra = _run_agg(feat.reshape(4 * N, DH), edges, zeros_h)
    nf = _run_dense(inter, c_inter, intra, c_intra, feat,
                    W_lr_0, W_ll_0, self_W_0, self_b_0)
    inter, intra = _run_agg(nf.reshape(4 * N, DH), edges, zeros_h)
    return _run_dense(inter, c_inter, intra, c_intra, nf,
                      W_lr_1, W_ll_1, self_W_1, self_b_1)
